# baseline (speedup 1.0000x reference)
import functools

import jax
import jax.numpy as jnp
import numpy as np
from jax.experimental import pallas as pl
from jax.experimental.pallas import tpu as pltpu

U = 64
BLK = 1024
SUB = 128
S2N = 5


def _halfswap(x):
    return jnp.concatenate([x[:, U:], x[:, :U]], axis=1)


def _prep_kernel(idx_ref, coef_ref, s0m_ref, s1m_ref, s2m_ref, wv_ref, *, p):
    s0m = s0m_ref[...]
    s1m = s1m_ref[...]
    s2m = s2m_ref[...]
    acc = jnp.zeros(s0m.shape, jnp.float32)
    for q in range(p):
        i0 = idx_ref[q, 0]
        i1 = idx_ref[q, 1]
        i2 = idx_ref[q, 2]
        hit = (s0m == i0) & (s1m == i1) & (s2m == i2)
        acc = acc + jnp.where(hit, coef_ref[q], 0.0)
    wv_ref[...] = acc


def _tp_kernel(wv_ref, x0_ref, x1_ref, out_ref, *, s0n, s1n, s2n):
    a_chunks = s0n // 2
    b_chunks = s1n // 2
    wrows = [wv_ref[j, :] for j in range(a_chunks * 2 * b_chunks * 5)]

    def body(x0t, x1t, out_r):
        for c in range(3):
            accd = None
            accs = None
            for a in range(a_chunks):
                x0c = x0t[:, a * 2 * U: (a + 1) * 2 * U]
                for sw in range(2):
                    xc = x0c if sw == 0 else _halfswap(x0c)
                    for b in range(b_chunks):
                        t = xc * x1t[:, b * 2 * U: (b + 1) * 2 * U]
                        j = (((a * 2 + sw) * b_chunks) + b) * 5
                        if c < 2:
                            d = t * wrows[j + c]
                            s = t * wrows[j + 2 + c]
                            accd = d if accd is None else accd + d
                            accs = s if accs is None else accs + s
                        else:
                            m = t * wrows[j + 4]
                            accd = m if accd is None else accd + m
            if c < 2:
                out_r[:, c * 2 * U: (c + 1) * 2 * U] = accd + _halfswap(accs)
            else:
                out_r[:, 4 * U:] = accd[:, :U] + accd[:, U:]

    for st in range(BLK // SUB):
        rows = pl.ds(st * SUB, SUB)
        body(x0_ref[rows, :], x1_ref[rows, :], out_ref.at[rows, :])


def _segment_maps(s0n, s1n):
    a_chunks, b_chunks = s0n // 2, s1n // 2
    rows = []
    for a in range(a_chunks):
        for sw in range(2):
            for b in range(b_chunks):
                f = (2 * a + sw, 2 * b)
                s = (2 * a + 1 - sw, 2 * b + 1)
                for first_s2, second_s2 in ((0, 1), (2, 3), (1, 0), (3, 2),
                                            (4, 4)):
                    rows.append((f + (first_s2,), s + (second_s2,)))
    n_rows = len(rows)
    maps = np.zeros((3, n_rows, 2 * U), np.int32)
    for j, (fh, sh) in enumerate(rows):
        for d in range(3):
            maps[d, j, :U] = fh[d]
            maps[d, j, U:] = sh[d]
    return maps


def kernel(x0, x1, path_coefficients, path_indices):
    n = x0.shape[0]
    s0n = x0.shape[1] // U
    s1n = x1.shape[1] // U
    s2n = S2N
    p = path_coefficients.shape[0]

    maps = _segment_maps(s0n, s1n)
    n_rows = maps.shape[1]

    wv = pl.pallas_call(
        functools.partial(_prep_kernel, p=p),
        in_specs=[
            pl.BlockSpec(memory_space=pltpu.SMEM),
            pl.BlockSpec(memory_space=pltpu.SMEM),
            pl.BlockSpec(memory_space=pltpu.VMEM),
            pl.BlockSpec(memory_space=pltpu.VMEM),
            pl.BlockSpec(memory_space=pltpu.VMEM),
        ],
        out_specs=pl.BlockSpec(memory_space=pltpu.VMEM),
        out_shape=jax.ShapeDtypeStruct((n_rows, 2 * U), jnp.float32),
    )(path_indices, path_coefficients,
      jnp.asarray(maps[0]), jnp.asarray(maps[1]), jnp.asarray(maps[2]))

    grid = (n // BLK,)
    fn = functools.partial(_tp_kernel, s0n=s0n, s1n=s1n, s2n=s2n)
    out = pl.pallas_call(
        fn,
        grid=grid,
        in_specs=[
            pl.BlockSpec((n_rows, 2 * U), lambda i: (0, 0)),
            pl.BlockSpec((BLK, s0n * U), lambda i: (i, 0)),
            pl.BlockSpec((BLK, s1n * U), lambda i: (i, 0)),
        ],
        out_specs=pl.BlockSpec((BLK, s2n * U), lambda i: (i, 0)),
        out_shape=jax.ShapeDtypeStruct((n, s2n * U), jnp.float32),
        compiler_params=pltpu.CompilerParams(
            dimension_semantics=("arbitrary",),
        ),
    )(wv, x0, x1)
    return out


# speedup vs baseline: 1.0084x; 1.0084x over previous
import functools

import jax
import jax.numpy as jnp
import numpy as np
from jax.experimental import pallas as pl
from jax.experimental.pallas import tpu as pltpu

U = 64
BLK = 1024
SUB = 128
S2N = 5


def _halfswap(x):
    return jnp.concatenate([x[:, U:], x[:, :U]], axis=1)


def _prep_kernel(idx_ref, coef_ref, s0m_ref, s1m_ref, s2m_ref, wv_ref, *, p):
    s0m = s0m_ref[...]
    s1m = s1m_ref[...]
    s2m = s2m_ref[...]
    acc = jnp.zeros(s0m.shape, jnp.float32)
    for q in range(p):
        i0 = idx_ref[q, 0]
        i1 = idx_ref[q, 1]
        i2 = idx_ref[q, 2]
        hit = (s0m == i0) & (s1m == i1) & (s2m == i2)
        acc = acc + jnp.where(hit, coef_ref[q], 0.0)
    wv_ref[...] = acc


def _tp_kernel(idx_ref, coef_ref, s0m_ref, s1m_ref, s2m_ref,
               x0_ref, x1_ref, out_ref, wv_ref, *, s0n, s1n, s2n, p):
    a_chunks = s0n // 2
    b_chunks = s1n // 2

    @pl.when(pl.program_id(0) == 0)
    def _():
        _prep_kernel(idx_ref, coef_ref, s0m_ref, s1m_ref, s2m_ref, wv_ref,
                     p=p)

    wrows = [wv_ref[j, :] for j in range(a_chunks * 2 * b_chunks * 5)]

    def body(x0t, x1t, out_r):
        for c in range(3):
            accd = None
            accs = None
            for a in range(a_chunks):
                x0c = x0t[:, a * 2 * U: (a + 1) * 2 * U]
                for sw in range(2):
                    xc = x0c if sw == 0 else _halfswap(x0c)
                    for b in range(b_chunks):
                        t = xc * x1t[:, b * 2 * U: (b + 1) * 2 * U]
                        j = (((a * 2 + sw) * b_chunks) + b) * 5
                        if c < 2:
                            d = t * wrows[j + c]
                            s = t * wrows[j + 2 + c]
                            accd = d if accd is None else accd + d
                            accs = s if accs is None else accs + s
                        else:
                            m = t * wrows[j + 4]
                            accd = m if accd is None else accd + m
            if c < 2:
                out_r[:, c * 2 * U: (c + 1) * 2 * U] = accd + _halfswap(accs)
            else:
                out_r[:, 4 * U:] = accd[:, :U] + accd[:, U:]

    for st in range(BLK // SUB):
        rows = pl.ds(st * SUB, SUB)
        body(x0_ref[rows, :], x1_ref[rows, :], out_ref.at[rows, :])


def _segment_maps(s0n, s1n):
    a_chunks, b_chunks = s0n // 2, s1n // 2
    rows = []
    for a in range(a_chunks):
        for sw in range(2):
            for b in range(b_chunks):
                f = (2 * a + sw, 2 * b)
                s = (2 * a + 1 - sw, 2 * b + 1)
                for first_s2, second_s2 in ((0, 1), (2, 3), (1, 0), (3, 2),
                                            (4, 4)):
                    rows.append((f + (first_s2,), s + (second_s2,)))
    n_rows = len(rows)
    maps = np.zeros((3, n_rows, 2 * U), np.int32)
    for j, (fh, sh) in enumerate(rows):
        for d in range(3):
            maps[d, j, :U] = fh[d]
            maps[d, j, U:] = sh[d]
    return maps


def kernel(x0, x1, path_coefficients, path_indices):
    n = x0.shape[0]
    s0n = x0.shape[1] // U
    s1n = x1.shape[1] // U
    s2n = S2N
    p = path_coefficients.shape[0]

    maps = _segment_maps(s0n, s1n)
    n_rows = maps.shape[1]

    grid = (n // BLK,)
    fn = functools.partial(_tp_kernel, s0n=s0n, s1n=s1n, s2n=s2n, p=p)
    out = pl.pallas_call(
        fn,
        grid=grid,
        in_specs=[
            pl.BlockSpec(memory_space=pltpu.SMEM),
            pl.BlockSpec(memory_space=pltpu.SMEM),
            pl.BlockSpec((n_rows, 2 * U), lambda i: (0, 0),
                         memory_space=pltpu.VMEM),
            pl.BlockSpec((n_rows, 2 * U), lambda i: (0, 0),
                         memory_space=pltpu.VMEM),
            pl.BlockSpec((n_rows, 2 * U), lambda i: (0, 0),
                         memory_space=pltpu.VMEM),
            pl.BlockSpec((BLK, s0n * U), lambda i: (i, 0)),
            pl.BlockSpec((BLK, s1n * U), lambda i: (i, 0)),
        ],
        out_specs=pl.BlockSpec((BLK, s2n * U), lambda i: (i, 0)),
        out_shape=jax.ShapeDtypeStruct((n, s2n * U), jnp.float32),
        scratch_shapes=[
            pltpu.VMEM((n_rows, 2 * U), jnp.float32),
        ],
        compiler_params=pltpu.CompilerParams(
            dimension_semantics=("arbitrary",),
        ),
    )(path_indices, path_coefficients,
      jnp.asarray(maps[0]), jnp.asarray(maps[1]), jnp.asarray(maps[2]),
      x0, x1)
    return out
